# baseline (speedup 1.0000x reference)
import jax
import jax.numpy as jnp
from jax import lax
from jax.experimental import pallas as pl
from jax.experimental.pallas import tpu as pltpu
from jax.experimental.pallas import tpu_sc as plsc

_NW = 32


def _sc_body(mask_hbm, t_hbm, out_hbm, mask_v, zero_v, sem):
    rows, d = t_hbm.shape
    rpw = rows // _NW
    wid = lax.axis_index("s") * 2 + lax.axis_index("c")
    base = wid * rpw

    pltpu.sync_copy(mask_hbm.at[pl.ds(base, rpw)], mask_v)

    def zinit(j, carry):
        zero_v[pl.ds(j * 16, 16)] = jnp.zeros((16,), jnp.float32)
        return carry

    lax.fori_loop(0, d // 16, zinit, 0)

    def body(g, carry):
        m16 = mask_v[pl.ds(g * 16, 16)]
        for j in range(16):
            row = base + g * 16 + j
            m = m16[j]

            @pl.when(m != 0)
            def _keep():
                pltpu.async_copy(t_hbm.at[row], out_hbm.at[row], sem)

            @pl.when(m == 0)
            def _zero():
                pltpu.async_copy(zero_v, out_hbm.at[row], sem)

        return carry

    lax.fori_loop(0, rpw // 16, body, 0)

    pltpu.make_async_copy(
        t_hbm.at[pl.ds(base, rpw)], out_hbm.at[pl.ds(base, rpw)], sem
    ).wait()


def kernel(tensor, mask):
    B, S, D = tensor.shape
    rows = B * S
    t2d = tensor.reshape(rows, D)
    m1d = mask.astype(jnp.int32).reshape(rows)
    rpw = rows // _NW

    kfn = pl.kernel(
        _sc_body,
        out_type=jax.ShapeDtypeStruct((rows, D), jnp.float32),
        mesh=plsc.VectorSubcoreMesh(core_axis_name="c", subcore_axis_name="s"),
        scratch_types=[
            pltpu.VMEM((rpw,), jnp.int32),
            pltpu.VMEM((D,), jnp.float32),
            pltpu.SemaphoreType.DMA,
        ],
    )
    return kfn(m1d, t2d).reshape(B, S, D)


# speedup vs baseline: 22.2208x; 22.2208x over previous
import jax
import jax.numpy as jnp
from jax import lax
from jax.experimental import pallas as pl
from jax.experimental.pallas import tpu as pltpu
from jax.experimental.pallas import tpu_sc as plsc

_NW = 32
_K = 16

_DNUMS = lax.GatherDimensionNumbers(
    offset_dims=(), collapsed_slice_dims=(0,), start_index_map=(0,))


def _g16(v, idx):
    return lax.gather(v, idx[:, None], dimension_numbers=_DNUMS,
                      slice_sizes=(1,),
                      mode=lax.GatherScatterMode.PROMISE_IN_BOUNDS)


def _prefix16(x, iota):
    for sh in (1, 2, 4, 8):
        g = _g16(x, jnp.maximum(iota - sh, 0))
        x = x + jnp.where(iota >= sh, g, 0)
    return x


def _rank_select(cs, tgt):
    src = jnp.zeros((16,), jnp.int32)
    for step in (8, 4, 2, 1):
        c = _g16(cs, src + (step - 1))
        src = src + jnp.where(c <= tgt, step, 0)
    return src


def _sc_body(mask_hbm, t_hbm, out_hbm, mask_v, kept_v, miss_v,
             buf_a, buf_b, zbuf, sem_g, sem_s, sem_z):
    rows, d = t_hbm.shape
    rpw = rows // _NW
    wid = lax.axis_index("s") * 2 + lax.axis_index("c")
    base = wid * rpw

    pltpu.sync_copy(mask_hbm.at[pl.ds(base, rpw)], mask_v)

    zv = jnp.zeros((16,), jnp.float32)

    def zinit_row(i, carry):
        def zinit_col(j, carry2):
            zbuf[i, pl.ds(j * 16, 16)] = zv
            return carry2
        return lax.fori_loop(0, d // 16, zinit_col, carry)

    lax.fori_loop(0, _K, zinit_row, 0)

    iota = lax.iota(jnp.int32, 16)

    def compact(g, carry):
        nk, nm = carry
        m16 = mask_v[pl.ds(g * 16, 16)]
        idxv = base + g * 16 + iota
        mi = jnp.where(m16 != 0, 1, 0)
        csk = _prefix16(mi, iota)
        ck = csk[15]
        csm = iota + 1 - csk
        cm = 16 - ck
        sk = jnp.where(iota < ck,
                       _g16(idxv, _rank_select(csk, iota)),
                       _g16(idxv, _rank_select(csm, iota - ck)))
        sm = jnp.where(iota < cm,
                       _g16(idxv, _rank_select(csm, iota)),
                       _g16(idxv, _rank_select(csk, iota - cm)))
        kept_v[pl.ds(nk, 16)] = sk
        miss_v[pl.ds(nm, 16)] = sm
        return (nk + ck, nm + cm)

    nk, nm = lax.fori_loop(0, rpw // 16, compact,
                           (jnp.int32(0), jnp.int32(0)))

    m0 = jnp.where(nm > 0, miss_v[pl.ds(0, 16)][0], base)
    padv = jnp.zeros((16,), jnp.int32) + m0
    kept_v[pl.ds(nk, 16)] = padv
    miss_v[pl.ds(nm, 16)] = padv

    nck = (nk + _K - 1) // _K
    ncz = (nm + _K - 1) // _K

    @pl.when(nck > 0)
    def _prime():
        pltpu.async_copy(t_hbm.at[kept_v[pl.ds(0, 16)]], buf_a, sem_g)

    def kbody(c, carry):
        pltpu.make_async_copy(t_hbm.at[pl.ds(0, _K)], buf_a, sem_g).wait()
        iv = kept_v[pl.ds(c * 16, 16)]
        even = (c % 2) == 0

        @pl.when(even)
        def _se():
            pltpu.async_copy(buf_a, out_hbm.at[iv], sem_s)

        @pl.when(~even)
        def _so():
            pltpu.async_copy(buf_b, out_hbm.at[iv], sem_s)

        @pl.when(c + 1 < nck)
        def _issue():
            @pl.when(c >= 1)
            def _w():
                pltpu.make_async_copy(
                    buf_a, out_hbm.at[pl.ds(0, _K)], sem_s).wait()
            iv2 = kept_v[pl.ds((c + 1) * 16, 16)]

            @pl.when(even)
            def _ge():
                pltpu.async_copy(t_hbm.at[iv2], buf_b, sem_g)

            @pl.when(~even)
            def _go():
                pltpu.async_copy(t_hbm.at[iv2], buf_a, sem_g)

        return carry

    lax.fori_loop(0, nck, kbody, 0)

    def sdrain(c, carry):
        pltpu.make_async_copy(buf_a, out_hbm.at[pl.ds(0, _K)], sem_s).wait()
        return carry

    lax.fori_loop(0, jnp.minimum(nck, 2), sdrain, 0)

    def zbody(c, carry):
        pltpu.async_copy(zbuf, out_hbm.at[miss_v[pl.ds(c * 16, 16)]], sem_z)
        return carry

    lax.fori_loop(0, ncz, zbody, 0)

    def zdrain(c, carry):
        pltpu.make_async_copy(zbuf, out_hbm.at[pl.ds(0, _K)], sem_z).wait()
        return carry

    lax.fori_loop(0, ncz, zdrain, 0)


def kernel(tensor, mask):
    B, S, D = tensor.shape
    rows = B * S
    t2d = tensor.reshape(rows, D)
    m1d = mask.astype(jnp.int32).reshape(rows)
    rpw = rows // _NW

    kfn = pl.kernel(
        _sc_body,
        out_type=jax.ShapeDtypeStruct((rows, D), jnp.float32),
        mesh=plsc.VectorSubcoreMesh(core_axis_name="c", subcore_axis_name="s"),
        scratch_types=[
            pltpu.VMEM((rpw,), jnp.int32),
            pltpu.VMEM((rpw + 16,), jnp.int32),
            pltpu.VMEM((rpw + 16,), jnp.int32),
            pltpu.VMEM((_K, D), jnp.float32),
            pltpu.VMEM((_K, D), jnp.float32),
            pltpu.VMEM((_K, D), jnp.float32),
            pltpu.SemaphoreType.DMA,
            pltpu.SemaphoreType.DMA,
            pltpu.SemaphoreType.DMA,
        ],
    )
    return kfn(m1d, t2d).reshape(B, S, D)
